# Initial kernel scaffold; baseline (speedup 1.0000x reference)
import jax
import jax.numpy as jnp
from jax.experimental import pallas as pl


def kernel(feat, feat_a, adj, graph_neigh, edge_index, Wl1, Wr1, att1, Wl2, Wr2, att2, Wb, bb):
    raise NotImplementedError("write your pallas kernel here")


# speedup vs baseline: 2.1537x; 2.1537x over previous
import functools
import jax
import jax.numpy as jnp
from jax import lax
from jax.experimental import pallas as pl
from jax.experimental.pallas import tpu as pltpu

N = 10000
DIN = 128
DOUT = 64

BM = 400


def _mm_body(a_ref, b_ref, o_ref):
    o_ref[...] = jnp.dot(a_ref[...], b_ref[...],
                         preferred_element_type=jnp.float32)


def _big_matmul(a, b):
    n = a.shape[0]
    c = b.shape[1]
    return pl.pallas_call(
        _mm_body,
        grid=(n // BM,),
        in_specs=[
            pl.BlockSpec((BM, n), lambda i: (i, 0)),
            pl.BlockSpec((n, c), lambda i: (0, 0)),
        ],
        out_specs=pl.BlockSpec((BM, c), lambda i: (i, 0)),
        out_shape=jax.ShapeDtypeStruct((n, c), jnp.float32),
    )(a, b)


def _readout_body(a_ref, b_ref, o_ref):
    ab = a_ref[...]
    vsum = jnp.dot(ab, b_ref[...], preferred_element_type=jnp.float32)
    rs = jnp.sum(ab, axis=1, keepdims=True)
    g = vsum / rs
    g1 = g[:, :DOUT]
    g2 = g[:, DOUT:]
    n1 = jnp.maximum(jnp.sqrt(jnp.sum(g1 * g1, axis=1, keepdims=True)), 1e-12)
    n2 = jnp.maximum(jnp.sqrt(jnp.sum(g2 * g2, axis=1, keepdims=True)), 1e-12)
    o_ref[...] = jax.nn.sigmoid(jnp.concatenate([g1 / n1, g2 / n2], axis=1))


def _readout(graph_neigh, embc):
    n = graph_neigh.shape[0]
    return pl.pallas_call(
        _readout_body,
        grid=(n // BM,),
        in_specs=[
            pl.BlockSpec((BM, n), lambda i: (i, 0)),
            pl.BlockSpec((n, 2 * DOUT), lambda i: (0, 0)),
        ],
        out_specs=pl.BlockSpec((BM, 2 * DOUT), lambda i: (i, 0)),
        out_shape=jax.ShapeDtypeStruct((n, 2 * DOUT), jnp.float32),
    )(graph_neigh, embc)


def _xform_body(x_ref, w_ref, o_ref):
    o_ref[...] = jnp.dot(x_ref[...], w_ref[...],
                         preferred_element_type=jnp.float32)


def _xform(x, wt):
    n, kdim = x.shape
    c = wt.shape[1]
    return pl.pallas_call(
        _xform_body,
        grid=(n // BM,),
        in_specs=[
            pl.BlockSpec((BM, kdim), lambda i: (i, 0)),
            pl.BlockSpec((kdim, c), lambda i: (0, 0)),
        ],
        out_specs=pl.BlockSpec((BM, c), lambda i: (i, 0)),
        out_shape=jax.ShapeDtypeStruct((n, c), jnp.float32),
    )(x, wt)


def _disc_body(emb_ref, g_ref, wb_ref, bb_ref, r_ref, ra_ref):
    bb = bb_ref[0, 0]
    emb = emb_ref[...]
    g = g_ref[...]
    e1, e2 = emb[:, :DOUT], emb[:, DOUT:]
    c1, c2 = g[:, :DOUT], g[:, DOUT:]
    wb = wb_ref[...]
    t1 = jnp.dot(e1, wb, preferred_element_type=jnp.float32)
    t2 = jnp.dot(e2, wb, preferred_element_type=jnp.float32)
    s11 = jnp.sum(t1 * c1, axis=1, keepdims=True) + bb
    s12 = jnp.sum(t2 * c1, axis=1, keepdims=True) + bb
    s21 = jnp.sum(t2 * c2, axis=1, keepdims=True) + bb
    s22 = jnp.sum(t1 * c2, axis=1, keepdims=True) + bb
    r_ref[...] = jax.nn.sigmoid(jnp.concatenate([s11, s12], axis=1))
    ra_ref[...] = jax.nn.sigmoid(jnp.concatenate([s21, s22], axis=1))


def _disc(embc, gc, wb, bb):
    n = embc.shape[0]
    return pl.pallas_call(
        _disc_body,
        grid=(n // BM,),
        in_specs=[
            pl.BlockSpec((BM, 2 * DOUT), lambda i: (i, 0)),
            pl.BlockSpec((BM, 2 * DOUT), lambda i: (i, 0)),
            pl.BlockSpec((DOUT, DOUT), lambda i: (0, 0)),
            pl.BlockSpec((1, 1), lambda i: (0, 0)),
        ],
        out_specs=[
            pl.BlockSpec((BM, 2), lambda i: (i, 0)),
            pl.BlockSpec((BM, 2), lambda i: (i, 0)),
        ],
        out_shape=[
            jax.ShapeDtypeStruct((n, 2), jnp.float32),
            jax.ShapeDtypeStruct((n, 2), jnp.float32),
        ],
    )(embc, gc, wb, bb.reshape(1, 1))


def _gat(xl, xr, src, dst, att, n):
    e = jax.nn.leaky_relu(xl[src] + xr[dst], negative_slope=0.2)
    alpha = jnp.sum(e * att, axis=-1)
    m = jnp.max(alpha)
    ex = jnp.exp(alpha - m)
    num = jax.ops.segment_sum(xl[src] * ex[:, None], dst, num_segments=n)
    den = jax.ops.segment_sum(ex, dst, num_segments=n)
    return num / (den[:, None] + 1e-16)


def kernel(feat, feat_a, adj, graph_neigh, edge_index, Wl1, Wr1, att1,
           Wl2, Wr2, att2, Wb, bb):
    src = edge_index[0]
    dst = edge_index[1]

    w1t = jnp.concatenate([Wl1, Wr1], axis=0).T
    xlr1 = _xform(feat, w1t)
    xlr1a = _xform(feat_a, w1t)
    xl1, xr1 = xlr1[:, :DOUT], xlr1[:, DOUT:]
    xl1a, xr1a = xlr1a[:, :DOUT], xlr1a[:, DOUT:]

    g1 = _gat(xl1, xr1, src, dst, att1, N)
    g1a = _gat(xl1a, xr1a, src, dst, att1, N)

    zc = _big_matmul(adj, jnp.concatenate([g1, g1a], axis=1))
    z = zc[:, :DOUT]

    w2t = jnp.concatenate([Wl2, Wr2], axis=0).T
    xlr2 = _xform(z, w2t)
    xl2, xr2 = xlr2[:, :DIN], xlr2[:, DIN:]

    h_pre = _gat(xl2, xr2, src, dst, att2, N)
    h = _big_matmul(adj, h_pre)

    embc = jnp.maximum(zc, 0.0)
    gc = _readout(graph_neigh, embc)
    ret, ret_a = _disc(embc, gc, Wb, bb)
    return (z, h, ret, ret_a)
